# Initial kernel scaffold; baseline (speedup 1.0000x reference)
import functools
import math

import numpy as np
import jax
import jax.numpy as jnp
from jax import lax
from jax.experimental import pallas as pl
from jax.experimental.pallas import tpu as pltpu
from jax.experimental.pallas import tpu_sc as plsc

D_MODEL = 512
SEQ_LEN = 8192
SCALE = math.sqrt(D_MODEL)

NUM_CORES = 2
NUM_SUBCORES = 16
NUM_WORKERS = NUM_CORES * NUM_SUBCORES
ROWS_PER_WORKER = SEQ_LEN // NUM_WORKERS
CHUNK = 32
NUM_CHUNKS = ROWS_PER_WORKER // CHUNK
LANES = 16


def _pe_table() -> np.ndarray:
    position = np.arange(SEQ_LEN, dtype=np.float32)[:, None]
    div_term = np.exp(
        np.arange(0, D_MODEL, 2, dtype=np.float32) * (-math.log(10000.0) / D_MODEL)
    )
    pe = np.zeros((SEQ_LEN, D_MODEL), dtype=np.float32)
    pe[:, 0::2] = np.sin(position * div_term)
    pe[:, 1::2] = np.cos(position * div_term)
    return pe


_PE = _pe_table()

_MESH = plsc.VectorSubcoreMesh(core_axis_name="c", subcore_axis_name="s")


@functools.partial(
    pl.kernel,
    mesh=_MESH,
    out_type=jax.ShapeDtypeStruct((SEQ_LEN, D_MODEL), jnp.float32),
    scratch_types=[
        pltpu.VMEM((ROWS_PER_WORKER,), jnp.int32),
        pltpu.VMEM((2, CHUNK, D_MODEL), jnp.float32),
        pltpu.VMEM((2, CHUNK, D_MODEL), jnp.float32),
        pltpu.VMEM((2, CHUNK, D_MODEL), jnp.float32),
        pltpu.SemaphoreType.DMA,
        pltpu.SemaphoreType.DMA,
        pltpu.SemaphoreType.DMA,
        pltpu.SemaphoreType.DMA,
        pltpu.SemaphoreType.DMA,
        pltpu.SemaphoreType.DMA,
    ],
)
def _sc_embed_pe(idx_hbm, pe_hbm, table_hbm, out_hbm,
                 idx_v, rows_v, pe_v, out_v,
                 g_sem0, g_sem1, p_sem0, p_sem1, o_sem0, o_sem1):
    wid = lax.axis_index("s") * NUM_CORES + lax.axis_index("c")
    base = wid * ROWS_PER_WORKER

    g_sems = (g_sem0, g_sem1)
    p_sems = (p_sem0, p_sem1)
    o_sems = (o_sem0, o_sem1)

    pltpu.sync_copy(idx_hbm.at[pl.ds(base, ROWS_PER_WORKER)], idx_v)

    def start_chunk(g):
        b = g % 2
        idx_slice = idx_v.at[pl.ds(g * CHUNK, CHUNK)]
        gather = pltpu.async_copy(table_hbm.at[idx_slice], rows_v.at[b], g_sems[b])
        pe_cp = pltpu.async_copy(
            pe_hbm.at[pl.ds(base + g * CHUNK, CHUNK)], pe_v.at[b], p_sems[b]
        )
        return gather, pe_cp

    in_flight = {0: start_chunk(0), 1: start_chunk(1)}
    out_flight = {}

    for g in range(NUM_CHUNKS):
        b = g % 2
        if g >= 2:
            out_flight.pop(g - 2).wait()
        gather, pe_cp = in_flight.pop(g)
        gather.wait()
        pe_cp.wait()

        def row_body(r):
            for c in range(D_MODEL // LANES):
                sl = pl.ds(c * LANES, LANES)
                out_v[b, r, sl] = rows_v[b, r, sl] * SCALE + pe_v[b, r, sl]

        lax.fori_loop(0, CHUNK, lambda r, _: (row_body(r), 0)[1], 0)

        out_flight[g] = pltpu.async_copy(
            out_v.at[b], out_hbm.at[pl.ds(base + g * CHUNK, CHUNK)], o_sems[b]
        )
        if g + 2 < NUM_CHUNKS:
            in_flight[g + 2] = start_chunk(g + 2)

    out_flight.pop(NUM_CHUNKS - 2).wait()
    out_flight.pop(NUM_CHUNKS - 1).wait()


@jax.jit
def _run(inputs, table):
    pe = jnp.asarray(_PE)
    return _sc_embed_pe(inputs.astype(jnp.int32), pe, table)


def kernel(inputs, table):
    return _run(inputs, table)


# speedup vs baseline: 2.6304x; 2.6304x over previous
import functools
import math

import numpy as np
import jax
import jax.numpy as jnp
from jax import lax
from jax.experimental import pallas as pl
from jax.experimental.pallas import tpu as pltpu
from jax.experimental.pallas import tpu_sc as plsc

D_MODEL = 512
SEQ_LEN = 8192
SCALE = math.sqrt(D_MODEL)

NUM_CORES = 2
NUM_SUBCORES = 16
NUM_WORKERS = NUM_CORES * NUM_SUBCORES
ROWS_PER_WORKER = SEQ_LEN // NUM_WORKERS
CHUNK = 32
NUM_CHUNKS = ROWS_PER_WORKER // CHUNK
LANES = 16


def _pe_table() -> np.ndarray:
    position = np.arange(SEQ_LEN, dtype=np.float64)[:, None]
    div_term = np.exp(
        np.arange(0, D_MODEL, 2, dtype=np.float32).astype(np.float64)
        * (-math.log(10000.0) / D_MODEL)
    )
    pe = np.zeros((SEQ_LEN, D_MODEL), dtype=np.float32)
    pe[:, 0::2] = np.sin(position * div_term).astype(np.float32)
    pe[:, 1::2] = np.cos(position * div_term).astype(np.float32)
    return pe


_PE = _pe_table()

_MESH = plsc.VectorSubcoreMesh(core_axis_name="c", subcore_axis_name="s")


@functools.partial(
    pl.kernel,
    mesh=_MESH,
    out_type=jax.ShapeDtypeStruct((SEQ_LEN, D_MODEL), jnp.float32),
    scratch_types=[
        pltpu.VMEM((ROWS_PER_WORKER,), jnp.int32),
        pltpu.VMEM((2, CHUNK, D_MODEL), jnp.float32),
        pltpu.VMEM((2, CHUNK, D_MODEL), jnp.float32),
        pltpu.VMEM((2, CHUNK, D_MODEL), jnp.float32),
        pltpu.SemaphoreType.DMA,
        pltpu.SemaphoreType.DMA,
        pltpu.SemaphoreType.DMA,
        pltpu.SemaphoreType.DMA,
        pltpu.SemaphoreType.DMA,
        pltpu.SemaphoreType.DMA,
    ],
)
def _sc_embed_pe(idx_hbm, pe_hbm, table_hbm, out_hbm,
                 idx_v, rows_v, pe_v, out_v,
                 g_sem0, g_sem1, p_sem0, p_sem1, o_sem0, o_sem1):
    wid = lax.axis_index("s") * NUM_CORES + lax.axis_index("c")
    base = wid * ROWS_PER_WORKER

    g_sems = (g_sem0, g_sem1)
    p_sems = (p_sem0, p_sem1)
    o_sems = (o_sem0, o_sem1)

    pltpu.sync_copy(idx_hbm.at[pl.ds(base, ROWS_PER_WORKER)], idx_v)

    def start_chunk(g):
        b = g % 2
        idx_slice = idx_v.at[pl.ds(g * CHUNK, CHUNK)]
        gather = pltpu.async_copy(table_hbm.at[idx_slice], rows_v.at[b], g_sems[b])
        pe_cp = pltpu.async_copy(
            pe_hbm.at[pl.ds(base + g * CHUNK, CHUNK)], pe_v.at[b], p_sems[b]
        )
        return gather, pe_cp

    in_flight = {0: start_chunk(0), 1: start_chunk(1)}
    out_flight = {}

    for g in range(NUM_CHUNKS):
        b = g % 2
        if g >= 2:
            out_flight.pop(g - 2).wait()
        gather, pe_cp = in_flight.pop(g)
        gather.wait()
        pe_cp.wait()

        def row_body(r):
            for c in range(D_MODEL // LANES):
                sl = pl.ds(c * LANES, LANES)
                out_v[b, r, sl] = rows_v[b, r, sl] * SCALE + pe_v[b, r, sl]

        lax.fori_loop(0, CHUNK, lambda r, _: (row_body(r), 0)[1], 0)

        out_flight[g] = pltpu.async_copy(
            out_v.at[b], out_hbm.at[pl.ds(base + g * CHUNK, CHUNK)], o_sems[b]
        )
        if g + 2 < NUM_CHUNKS:
            in_flight[g + 2] = start_chunk(g + 2)

    out_flight.pop(NUM_CHUNKS - 2).wait()
    out_flight.pop(NUM_CHUNKS - 1).wait()


@jax.jit
def _run(inputs, table):
    pe = jnp.asarray(_PE)
    return _sc_embed_pe(inputs.astype(jnp.int32), pe, table)


def kernel(inputs, table):
    return _run(inputs, table)
